# Initial kernel scaffold; baseline (speedup 1.0000x reference)
import jax
import jax.numpy as jnp
from jax.experimental import pallas as pl


def kernel(nodes, neigh_idx, neigh_feats, feat_table, W, a_param, detaching_weight):
    raise NotImplementedError("write your pallas kernel here")


# speedup vs baseline: 11.0854x; 11.0854x over previous
import functools

import jax
import jax.numpy as jnp
import numpy as np
from jax import lax
from jax.experimental import pallas as pl
from jax.experimental.pallas import tpu as pltpu
from jax.experimental.pallas import tpu_sc as plsc

F = 128
D = 128
K = 16
LANES = 16
NSUB = F // LANES
NEG_SLOPE = np.float32(0.2)


@functools.lru_cache(maxsize=None)
def _make_sc_attend(B: int, C: int):
  info = plsc.get_sparse_core_info()
  nw = info.num_cores * info.num_subcores
  rpw = B // nw
  nch = rpw // C
  mesh = plsc.VectorSubcoreMesh(core_axis_name="c", subcore_axis_name="s")

  @functools.partial(
      pl.kernel,
      out_type=[
          jax.ShapeDtypeStruct((B, F), jnp.float32),
          jax.ShapeDtypeStruct((B, F), jnp.float32),
          jax.ShapeDtypeStruct((B, LANES), jnp.float32),
      ],
      mesh=mesh,
      compiler_params=pltpu.CompilerParams(needs_layout_passes=False),
      scratch_types=[
          pltpu.VMEM((C,), jnp.int32),
          pltpu.VMEM((C * K,), jnp.int32),
          pltpu.VMEM((C, F), jnp.float32),
          pltpu.VMEM((C * K, F), jnp.float32),
          pltpu.VMEM((C, F), jnp.float32),
          pltpu.VMEM((C, LANES), jnp.float32),
          pltpu.VMEM((2, F), jnp.float32),
          pltpu.SemaphoreType.DMA,
          pltpu.SemaphoreType.DMA,
      ],
  )
  def sc_attend(table, nodes, nidx, w12, emb2_o, s_o, rs_o,
                nodes_v, nidx_v, e2b, nbb, sb, rsb, wv, sem1, sem2):
    wid = lax.axis_index("s") * info.num_cores + lax.axis_index("c")
    base0 = wid * rpw
    pltpu.sync_copy(w12, wv)
    w1v = [wv[0, pl.ds(LANES * j, LANES)] for j in range(NSUB)]
    w2v = [wv[1, pl.ds(LANES * j, LANES)] for j in range(NSUB)]

    def chunk_body(g, carry):
      base = base0 + g * C
      pltpu.sync_copy(nodes.at[pl.ds(base, C)], nodes_v)
      pltpu.sync_copy(nidx.at[pl.ds(base * K, C * K)], nidx_v)
      cp1 = pltpu.async_copy(table.at[nodes_v], e2b, sem1)
      cp2 = pltpu.async_copy(table.at[nidx_v], nbb, sem2)
      cp1.wait()
      cp2.wait()

      def row_body(r, rcarry):
        ev = [e2b[r, pl.ds(LANES * j, LANES)] for j in range(NSUB)]
        p = ev[0] * w1v[0]
        for j in range(1, NSUB):
          p = p + ev[j] * w1v[j]
        c2 = jnp.sum(p)
        rsv = jnp.zeros((LANES,), jnp.float32)
        acc = [jnp.zeros((LANES,), jnp.float32) for _ in range(NSUB)]
        for k in range(K):
          nv = [nbb[r * K + k, pl.ds(LANES * j, LANES)] for j in range(NSUB)]
          d = nv[0] * w2v[0]
          for j in range(1, NSUB):
            d = d + nv[j] * w2v[j]
          logit = jnp.sum(d) + c2
          lr = jnp.where(logit >= 0, logit, logit * NEG_SLOPE)
          e = jnp.exp(jnp.full((LANES,), -lr, jnp.float32))
          rsv = rsv + e
          for j in range(NSUB):
            acc[j] = acc[j] + e * nv[j]
        for j in range(NSUB):
          sb[r, pl.ds(LANES * j, LANES)] = acc[j]
        rsb[r, :] = rsv
        return rcarry

      lax.fori_loop(0, C, row_body, 0)
      pltpu.sync_copy(e2b, emb2_o.at[pl.ds(base, C)])
      pltpu.sync_copy(sb, s_o.at[pl.ds(base, C)])
      pltpu.sync_copy(rsb, rs_o.at[pl.ds(base, C)])
      return carry

    lax.fori_loop(0, nch, chunk_body, 0)

  return sc_attend


def _tc_finish(emb2, s, rs, nf, W, wd1, wd2, wd3):
  B = emb2.shape[0]
  BM = 2048

  def body(e2_r, s_r, rs_r, nf_r, w_r, wd1_r, wd2_r, wd3_r, o_r):
    hp = jnp.dot(s_r[...], w_r[...], preferred_element_type=jnp.float32)
    hp = jnp.nan_to_num(hp)
    hp = jnp.nan_to_num(hp / rs_r[:, 0:1])
    acc = jnp.dot(e2_r[...], wd1_r[...], preferred_element_type=jnp.float32)
    acc = acc + jnp.dot(hp, wd2_r[...], preferred_element_type=jnp.float32)
    acc = acc + jnp.dot(nf_r[...], wd3_r[...], preferred_element_type=jnp.float32)
    o_r[...] = jnp.maximum(acc, 0.0)

  return pl.pallas_call(
      body,
      grid=(B // BM,),
      in_specs=[
          pl.BlockSpec((BM, F), lambda i: (i, 0)),
          pl.BlockSpec((BM, F), lambda i: (i, 0)),
          pl.BlockSpec((BM, LANES), lambda i: (i, 0)),
          pl.BlockSpec((BM, F), lambda i: (i, 0)),
          pl.BlockSpec((F, D), lambda i: (0, 0)),
          pl.BlockSpec((F, D), lambda i: (0, 0)),
          pl.BlockSpec((D, D), lambda i: (0, 0)),
          pl.BlockSpec((F, D), lambda i: (0, 0)),
      ],
      out_specs=pl.BlockSpec((BM, D), lambda i: (i, 0)),
      out_shape=jax.ShapeDtypeStruct((B, D), jnp.float32),
  )(emb2, s, rs, nf, W, wd1, wd2, wd3)


@jax.jit
def kernel(nodes, neigh_idx, neigh_feats, feat_table, W, a_param,
           detaching_weight):
  B, _ = neigh_idx.shape
  nodes32 = nodes.astype(jnp.int32)
  nidx_flat = neigh_idx.reshape(-1).astype(jnp.int32)
  w12 = a_param.reshape(2, D).astype(jnp.float32) @ W.T
  sc = _make_sc_attend(B, 8)
  emb2, s, rs = sc(feat_table, nodes32, nidx_flat, w12)
  wd1 = detaching_weight[:F]
  wd2 = detaching_weight[F:F + D]
  wd3 = detaching_weight[F + D:]
  return _tc_finish(emb2, s, rs, neigh_feats, W, wd1, wd2, wd3)


# speedup vs baseline: 16.8685x; 1.5217x over previous
import functools

import jax
import jax.numpy as jnp
import numpy as np
from jax import lax
from jax.experimental import pallas as pl
from jax.experimental.pallas import tpu as pltpu
from jax.experimental.pallas import tpu_sc as plsc

F = 128
D = 128
K = 16
LANES = 16
NSUB = F // LANES
NEG_SLOPE = np.float32(0.2)


@functools.lru_cache(maxsize=None)
def _make_sc_attend(B: int, C: int, NBUF: int):
  info = plsc.get_sparse_core_info()
  nw = info.num_cores * info.num_subcores
  rpw = B // nw
  nch = rpw // C
  assert nch % NBUF == 0
  mesh = plsc.VectorSubcoreMesh(core_axis_name="c", subcore_axis_name="s")

  @functools.partial(
      pl.kernel,
      out_type=[
          jax.ShapeDtypeStruct((B, F), jnp.float32),
          jax.ShapeDtypeStruct((B, F), jnp.float32),
          jax.ShapeDtypeStruct((B, LANES), jnp.float32),
      ],
      mesh=mesh,
      compiler_params=pltpu.CompilerParams(needs_layout_passes=False),
      scratch_types=[
          pltpu.VMEM((nch, C), jnp.int32),
          pltpu.VMEM((nch, C * K), jnp.int32),
          pltpu.VMEM((NBUF, C, F), jnp.float32),
          pltpu.VMEM((NBUF, C * K, F), jnp.float32),
          pltpu.VMEM((NBUF, C, F), jnp.float32),
          pltpu.VMEM((NBUF, C, LANES), jnp.float32),
          pltpu.VMEM((2, F), jnp.float32),
      ] + [pltpu.SemaphoreType.DMA] * (3 * NBUF),
  )
  def sc_attend(table, nodes2d, nidx2d, w12, emb2_o, s_o, rs_o,
                nodes_all, nidx_all, e2b, nbb, sb, rsb, wv, *sems):
    semg1 = sems[0:NBUF]
    semg2 = sems[NBUF:2 * NBUF]
    semw = sems[2 * NBUF:3 * NBUF]
    wid = lax.axis_index("s") * info.num_cores + lax.axis_index("c")
    base0 = wid * rpw
    row0 = wid * nch
    pltpu.sync_copy(w12, wv)
    pltpu.sync_copy(nodes2d.at[pl.ds(row0, nch)], nodes_all)
    pltpu.sync_copy(nidx2d.at[pl.ds(row0, nch)], nidx_all)
    w1v = [wv[0, pl.ds(LANES * j, LANES)] for j in range(NSUB)]
    w2v = [wv[1, pl.ds(LANES * j, LANES)] for j in range(NSUB)]

    def gather_cps(i, b):
      return (pltpu.make_async_copy(table.at[nodes_all.at[i]], e2b.at[b],
                                    semg1[b]),
              pltpu.make_async_copy(table.at[nidx_all.at[i]], nbb.at[b],
                                    semg2[b]))

    def wb_cps(i, b):
      base = base0 + i * C
      return (pltpu.make_async_copy(e2b.at[b], emb2_o.at[pl.ds(base, C)],
                                    semw[b]),
              pltpu.make_async_copy(sb.at[b], s_o.at[pl.ds(base, C)],
                                    semw[b]),
              pltpu.make_async_copy(rsb.at[b], rs_o.at[pl.ds(base, C)],
                                    semw[b]))

    def compute(b):
      def row_body(r, rcarry):
        ev = [e2b[b, r, pl.ds(LANES * j, LANES)] for j in range(NSUB)]
        p = ev[0] * w1v[0]
        for j in range(1, NSUB):
          p = p + ev[j] * w1v[j]
        c2 = jnp.sum(p)
        rsv = jnp.zeros((LANES,), jnp.float32)
        acc = [jnp.zeros((LANES,), jnp.float32) for _ in range(NSUB)]
        for k in range(K):
          nv = [nbb[b, r * K + k, pl.ds(LANES * j, LANES)]
                for j in range(NSUB)]
          d = nv[0] * w2v[0]
          for j in range(1, NSUB):
            d = d + nv[j] * w2v[j]
          logit = jnp.sum(d) + c2
          lr = jnp.where(logit >= 0, logit, logit * NEG_SLOPE)
          e = jnp.exp(jnp.full((LANES,), -lr, jnp.float32))
          rsv = rsv + e
          for j in range(NSUB):
            acc[j] = acc[j] + e * nv[j]
        for j in range(NSUB):
          sb[b, r, pl.ds(LANES * j, LANES)] = acc[j]
        rsb[b, r, :] = rsv
        return rcarry

      lax.fori_loop(0, C, row_body, 0)

    for b in range(NBUF):
      for cp in gather_cps(b, b):
        cp.start()

    def group(g, carry):
      for b in range(NBUF):
        i = g * NBUF + b
        for cp in gather_cps(i, b):
          cp.wait()
        compute(b)
        for cp in wb_cps(i, b):
          cp.start()
        inext = i + NBUF

        @pl.when(inext < nch)
        def _reissue():
          for cp in wb_cps(i, b):
            cp.wait()
          for cp in gather_cps(inext, b):
            cp.start()

      return carry

    lax.fori_loop(0, nch // NBUF, group, 0)
    for b in range(NBUF):
      for cp in wb_cps(nch - NBUF + b, b):
        cp.wait()

  return sc_attend


def _tc_finish(emb2, s, rs, nf, W, wd1, wd2, wd3):
  B = emb2.shape[0]
  BM = 2048

  def body(e2_r, s_r, rs_r, nf_r, w_r, wd1_r, wd2_r, wd3_r, o_r):
    hp = jnp.dot(s_r[...], w_r[...], preferred_element_type=jnp.float32)
    hp = jnp.nan_to_num(hp)
    hp = jnp.nan_to_num(hp / rs_r[:, 0:1])
    acc = jnp.dot(e2_r[...], wd1_r[...], preferred_element_type=jnp.float32)
    acc = acc + jnp.dot(hp, wd2_r[...], preferred_element_type=jnp.float32)
    acc = acc + jnp.dot(nf_r[...], wd3_r[...], preferred_element_type=jnp.float32)
    o_r[...] = jnp.maximum(acc, 0.0)

  return pl.pallas_call(
      body,
      grid=(B // BM,),
      in_specs=[
          pl.BlockSpec((BM, F), lambda i: (i, 0)),
          pl.BlockSpec((BM, F), lambda i: (i, 0)),
          pl.BlockSpec((BM, LANES), lambda i: (i, 0)),
          pl.BlockSpec((BM, F), lambda i: (i, 0)),
          pl.BlockSpec((F, D), lambda i: (0, 0)),
          pl.BlockSpec((F, D), lambda i: (0, 0)),
          pl.BlockSpec((D, D), lambda i: (0, 0)),
          pl.BlockSpec((F, D), lambda i: (0, 0)),
      ],
      out_specs=pl.BlockSpec((BM, D), lambda i: (i, 0)),
      out_shape=jax.ShapeDtypeStruct((B, D), jnp.float32),
  )(emb2, s, rs, nf, W, wd1, wd2, wd3)


@jax.jit
def kernel(nodes, neigh_idx, neigh_feats, feat_table, W, a_param,
           detaching_weight):
  B, _ = neigh_idx.shape
  C, NBUF = 8, 4
  nodes2d = nodes.astype(jnp.int32).reshape(B // C, C)
  nidx2d = neigh_idx.astype(jnp.int32).reshape(B // C, C * K)
  w12 = a_param.reshape(2, D).astype(jnp.float32) @ W.T
  sc = _make_sc_attend(B, C, NBUF)
  emb2, s, rs = sc(feat_table, nodes2d, nidx2d, w12)
  wd1 = detaching_weight[:F]
  wd2 = detaching_weight[F:F + D]
  wd3 = detaching_weight[F + D:]
  return _tc_finish(emb2, s, rs, neigh_feats, W, wd1, wd2, wd3)
